# Initial kernel scaffold; baseline (speedup 1.0000x reference)
import functools

import jax
import jax.numpy as jnp
from jax import lax
from jax.experimental import pallas as pl
from jax.experimental.pallas import tpu as pltpu
from jax.experimental.pallas import tpu_sc as plsc

B = 16384
NSTIM = 21
NPAIR = NSTIM * NSTIM
NPAIR_PAD = 448
L = 16
NC, NS = 2, 16
NW = NC * NS
BPW = B // NW
NGROUPS = BPW // L

_PAIRS = [(i, j) for i in range(8) for j in range(8) if j != i]


def _sqrt_f32(x):
    i = plsc.bitcast(x, jnp.int32)
    i = jnp.int32(0x5F3759DF) - jnp.right_shift(i, 1)
    y = plsc.bitcast(i, jnp.float32)
    for _ in range(3):
        y = y * (1.5 - 0.5 * x * y * y)
    return jnp.where(x > 0.0, x * y, 0.0)


def _splat(v, dtype=jnp.int32):
    return jnp.full((L,), v, dtype)


def _sc_body(g2_hbm, g8_hbm, tab_hbm, out1_hbm, out2_hbm,
             tab_v, s_v, idx2_v, idx8_v, out1_v, out2_v):
    wid = lax.axis_index("s") * NC + lax.axis_index("c")
    base = wid * BPW
    pltpu.sync_copy(tab_hbm, tab_v)
    pltpu.sync_copy(g2_hbm.at[pl.ds(base, BPW)], idx2_v)
    pltpu.sync_copy(g8_hbm.at[pl.ds(base, BPW)], idx8_v)

    def build_pairs(i, carry):
        p = jnp.minimum(i * L + lax.iota(jnp.int32, L), NPAIR - 1)
        q = p // NSTIM
        r = p - q * NSTIM
        q3, r3 = q * 3, r * 3
        d2 = None
        for d in range(3):
            diff = (plsc.load_gather(tab_v, [q3 + d])
                    - plsc.load_gather(tab_v, [r3 + d]))
            d2 = diff * diff if d2 is None else d2 + diff * diff
        s = jnp.exp(-10.0 * _sqrt_f32(d2)) + 0.001
        s_v[pl.ds(i * L, L)] = s
        return carry

    lax.fori_loop(0, NPAIR_PAD // L, build_pairs, 0)

    def group(g, carry):
        rows = g * L + lax.iota(jnp.int32, L)
        q = plsc.load_gather(idx2_v, [rows, _splat(0)]) * NSTIM
        s1 = plsc.load_gather(
            s_v, [q + plsc.load_gather(idx2_v, [rows, _splat(1)])])
        s2 = plsc.load_gather(
            s_v, [q + plsc.load_gather(idx2_v, [rows, _splat(2)])])
        inv = 1.0 / (s1 + s2)
        plsc.store_scatter(out1_v, [rows, _splat(0)], s1 * inv)
        plsc.store_scatter(out1_v, [rows, _splat(1)], s2 * inv)
        q = plsc.load_gather(idx8_v, [rows, _splat(0)]) * NSTIM
        s = [plsc.load_gather(
                s_v, [q + plsc.load_gather(idx8_v, [rows, _splat(j + 1)])])
             for j in range(8)]
        tot = s[0]
        for j in range(1, 8):
            tot = tot + s[j]
        invt = 1.0 / tot
        a = [(s[i] * invt) / (tot - s[i]) for i in range(8)]
        for k, (i, j) in enumerate(_PAIRS):
            plsc.store_scatter(out2_v, [rows, _splat(k)], a[i] * s[j])
        return carry

    lax.fori_loop(0, NGROUPS, group, 0)

    pltpu.sync_copy(out1_v, out1_hbm.at[pl.ds(base, BPW)])
    pltpu.sync_copy(out2_v, out2_hbm.at[pl.ds(base, BPW)])


@functools.cache
def _build():
    mesh = plsc.VectorSubcoreMesh(
        core_axis_name="c", subcore_axis_name="s",
        num_cores=NC, num_subcores=NS)
    return pl.kernel(
        _sc_body,
        out_type=(jax.ShapeDtypeStruct((B, 2), jnp.float32),
                  jax.ShapeDtypeStruct((B, 56), jnp.float32)),
        mesh=mesh,
        scratch_types=[
            pltpu.VMEM((64,), jnp.float32),
            pltpu.VMEM((NPAIR_PAD,), jnp.float32),
            pltpu.VMEM((BPW, 3), jnp.int32),
            pltpu.VMEM((BPW, 9), jnp.int32),
            pltpu.VMEM((BPW, 2), jnp.float32),
            pltpu.VMEM((BPW, 56), jnp.float32),
        ],
    )


def kernel(given2rank1_stimulus_set, given8rank2_stimulus_set, percept_table):
    tab_flat = jnp.pad(percept_table.reshape(-1), (0, 64 - 3 * NSTIM))
    return _build()(given2rank1_stimulus_set, given8rank2_stimulus_set,
                    tab_flat)


# speedup vs baseline: 11.3445x; 11.3445x over previous
import functools

import jax
import jax.numpy as jnp
from jax import lax
from jax.experimental import pallas as pl
from jax.experimental.pallas import tpu as pltpu
from jax.experimental.pallas import tpu_sc as plsc

B = 16384
NSTIM = 21
NPAIR = NSTIM * NSTIM
NPAIR_PAD = 448
L = 16
NC, NS = 2, 16
NW = NC * NS
BPW = B // NW
NGROUPS = BPW // L

_PAIRS = [(i, j) for i in range(8) for j in range(8) if j != i]


def _sqrt_f32(x):
    i = plsc.bitcast(x, jnp.int32)
    i = jnp.int32(0x5F3759DF) - jnp.right_shift(i, 1)
    y = plsc.bitcast(i, jnp.float32)
    for _ in range(3):
        y = y * (1.5 - 0.5 * x * y * y)
    return jnp.where(x > 0.0, x * y, 0.0)


def _splat(v, dtype=jnp.int32):
    return jnp.full((L,), v, dtype)


def _sc_body(g2_hbm, g8_hbm, tab_hbm, out1_hbm, out2_hbm,
             tab_v, s_v, idx2_v, idx8_v, out1_v, out2_v):
    wid = lax.axis_index("s") * NC + lax.axis_index("c")
    pltpu.sync_copy(tab_hbm, tab_v)
    pltpu.sync_copy(g2_hbm.at[pl.ds(wid * (BPW * 3), BPW * 3)], idx2_v)
    pltpu.sync_copy(g8_hbm.at[pl.ds(wid * (BPW * 9), BPW * 9)], idx8_v)

    def build_pairs(i, carry):
        p = jnp.minimum(i * L + lax.iota(jnp.int32, L), NPAIR - 1)
        q = jnp.right_shift(p * 3121, 16)
        r = p - q * NSTIM
        q3, r3 = q * 3, r * 3
        d2 = None
        for d in range(3):
            diff = (plsc.load_gather(tab_v, [q3 + d])
                    - plsc.load_gather(tab_v, [r3 + d]))
            d2 = diff * diff if d2 is None else d2 + diff * diff
        s_v[pl.ds(i * L, L)] = jnp.exp(-10.0 * _sqrt_f32(d2)) + 0.001
        return carry

    lax.fori_loop(0, NPAIR_PAD // L, build_pairs, 0)

    def group(g, carry):
        rows = g * L + lax.iota(jnp.int32, L)
        r3 = rows * 3
        q = plsc.load_gather(idx2_v, [r3]) * NSTIM
        s1 = plsc.load_gather(s_v, [q + plsc.load_gather(idx2_v, [r3 + 1])])
        s2 = plsc.load_gather(s_v, [q + plsc.load_gather(idx2_v, [r3 + 2])])
        inv = 1.0 / (s1 + s2)
        r2 = rows * 2
        plsc.store_scatter(out1_v, [r2], s1 * inv)
        plsc.store_scatter(out1_v, [r2 + 1], s2 * inv)
        r9 = rows * 9
        q = plsc.load_gather(idx8_v, [r9]) * NSTIM
        s = [plsc.load_gather(s_v, [q + plsc.load_gather(idx8_v, [r9 + j + 1])])
             for j in range(8)]
        tot = s[0]
        for j in range(1, 8):
            tot = tot + s[j]
        invt = 1.0 / tot
        a = [(s[i] * invt) / (tot - s[i]) for i in range(8)]
        r56 = rows * 56
        for k, (i, j) in enumerate(_PAIRS):
            plsc.store_scatter(out2_v, [r56 + k], a[i] * s[j])
        return carry

    lax.fori_loop(0, NGROUPS, group, 0)

    pltpu.sync_copy(out1_v, out1_hbm.at[pl.ds(wid * (BPW * 2), BPW * 2)])
    pltpu.sync_copy(out2_v, out2_hbm.at[pl.ds(wid * (BPW * 56), BPW * 56)])


@functools.cache
def _build():
    mesh = plsc.VectorSubcoreMesh(
        core_axis_name="c", subcore_axis_name="s",
        num_cores=NC, num_subcores=NS)
    return pl.kernel(
        _sc_body,
        out_type=(jax.ShapeDtypeStruct((B * 2,), jnp.float32),
                  jax.ShapeDtypeStruct((B * 56,), jnp.float32)),
        mesh=mesh,
        compiler_params=pltpu.CompilerParams(
            needs_layout_passes=False, use_tc_tiling_on_sc=False),
        scratch_types=[
            pltpu.VMEM((64,), jnp.float32),
            pltpu.VMEM((NPAIR_PAD,), jnp.float32),
            pltpu.VMEM((BPW * 3,), jnp.int32),
            pltpu.VMEM((BPW * 9,), jnp.int32),
            pltpu.VMEM((BPW * 2,), jnp.float32),
            pltpu.VMEM((BPW * 56,), jnp.float32),
        ],
    )


def kernel(given2rank1_stimulus_set, given8rank2_stimulus_set, percept_table):
    tab_flat = jnp.pad(percept_table.reshape(-1), (0, 64 - 3 * NSTIM))
    out1, out2 = _build()(given2rank1_stimulus_set.reshape(-1),
                          given8rank2_stimulus_set.reshape(-1), tab_flat)
    return (out1.reshape(B, 2), out2.reshape(B, 56))
